# baseline (speedup 1.0000x reference)
import functools

import jax
import jax.numpy as jnp
from jax import lax
from jax.experimental import pallas as pl
from jax.experimental.pallas import tpu as pltpu
from jax.experimental.pallas import tpu_sc as plsc

_STRIDE = 4
_BS = 32
_NPTS = 17
_H = 128
_W = 128
_FLAT = _H * _W
_NCH = 2 * _NPTS
_PAD = 48


_ROWS = _BS * _NPTS
_BR = 68


def _tc_argmax_body(s_ref, o_ref):
    s = s_ref[0]
    m = jnp.max(s, axis=1, keepdims=True)
    iota = lax.broadcasted_iota(jnp.int32, (_BR, _FLAT), 1)
    cand = jnp.where(s == m, iota, jnp.int32(_FLAT))
    idx = jnp.min(cand, axis=1, keepdims=True)
    o_ref[0] = jnp.broadcast_to(idx, (_BR, 128))


def _tc_argmax(score_flat):
    g = _ROWS // _BR
    out = pl.pallas_call(
        _tc_argmax_body,
        grid=(g,),
        in_specs=[pl.BlockSpec((1, _BR, _FLAT), lambda i: (i, 0, 0))],
        out_specs=pl.BlockSpec((1, _BR, 128), lambda i: (i, 0, 0)),
        out_shape=jax.ShapeDtypeStruct((g, _BR, 128), jnp.int32),
    )(score_flat.reshape(g, _BR, _FLAT))
    return out


def _sc_gather_body(idx_hbm, offtab_hbm, out_hbm,
                    ptrs_v, iv_v, offidx_v, off_v, out_v, sem):
    b = lax.axis_index("s") * 2 + lax.axis_index("c")
    for base in (0, 16, 32):
        jv = lax.iota(jnp.int32, 16) + base
        ptrs_v[pl.ds(base, 16)] = (b * _NPTS + (jv >> 1)) * 128
    pltpu.async_copy(idx_hbm.at[ptrs_v], iv_v, sem).wait()
    for base in (0, 16, 32):
        jv = lax.iota(jnp.int32, 16) + base
        iv = iv_v[pl.ds(base, 16)]
        yv = iv >> 7
        xv = iv & (_W - 1)
        oidx = ((b * _NCH + jv) * _H + yv) * _W + xv
        offidx_v[pl.ds(base, 16)] = jnp.where(jv < _NCH, oidx, 0)
    pltpu.async_copy(offtab_hbm.at[offidx_v], off_v, sem).wait()
    for base in (0, 16, 32):
        jv = lax.iota(jnp.int32, 16) + base
        iv = iv_v[pl.ds(base, 16)]
        yv = iv >> 7
        xv = iv & (_W - 1)
        coarse = jnp.where((jv & 1) == 0, yv, xv).astype(jnp.float32)
        off = off_v[pl.ds(base, 16)]
        out_v[pl.ds(base, 16)] = (coarse + off) * float(_STRIDE)
    pltpu.sync_copy(out_v, out_hbm.at[pl.ds(b * _PAD, _PAD)])


def _sc_gather(idx_flat, offset_flat):
    mesh = plsc.VectorSubcoreMesh(core_axis_name="c", subcore_axis_name="s")
    f = functools.partial(
        pl.kernel,
        mesh=mesh,
        out_type=jax.ShapeDtypeStruct((_BS * _PAD,), jnp.float32),
        scratch_types=[
            pltpu.VMEM((_PAD,), jnp.int32),
            pltpu.VMEM((_PAD,), jnp.int32),
            pltpu.VMEM((_PAD,), jnp.int32),
            pltpu.VMEM((_PAD,), jnp.float32),
            pltpu.VMEM((_PAD,), jnp.float32),
            pltpu.SemaphoreType.DMA,
        ],
    )(_sc_gather_body)
    return f(idx_flat, offset_flat)


def kernel(score_map, offset_map):
    score_flat = score_map.reshape(_ROWS, _FLAT)
    offset_flat = offset_map.reshape(_BS * _NCH * _FLAT)
    idx_tab = _tc_argmax(score_flat)
    out = _sc_gather(idx_tab.reshape(_ROWS * 128), offset_flat)
    return out.reshape(_BS, _PAD)[:, : _NCH].reshape(_BS, _NPTS, 2)
